# baseline (speedup 1.0000x reference)
import functools

import jax
import jax.numpy as jnp
from jax import lax
from jax.experimental import pallas as pl
from jax.experimental.pallas import tpu as pltpu
from jax.experimental.pallas import tpu_sc as plsc

_B = 16384
_D = 32
_NW = 32
_BPW = _B // _NW
_CH = 128
_NCHUNK = _BPW // _CH
_L = 16


def _pmf_body(u_hbm, i_hbm, uemb_hbm, iemb_hbm, out_hbm,
              idx_u, idx_i, rows_u, rows_i, out_v, sem_u, sem_i):
    wid = lax.axis_index("s") * 2 + lax.axis_index("c")

    pltpu.sync_copy(u_hbm.at[wid], idx_u)
    pltpu.sync_copy(i_hbm.at[wid], idx_i)

    copies = []
    for j in range(_NCHUNK):
        sl = pl.ds(j * _CH, _CH)
        copies.append(
            pltpu.async_copy(uemb_hbm.at[idx_u.at[j]], rows_u.at[sl], sem_u))
        copies.append(
            pltpu.async_copy(iemb_hbm.at[idx_i.at[j]], rows_i.at[sl], sem_i))
    for c in copies:
        c.wait()

    lane = lax.iota(jnp.int32, _L)

    def group_body(g, carry):
        b0 = g * _L
        acc = jnp.zeros((_L,), jnp.float32)
        for k in range(_L):
            b = b0 + k
            u0 = rows_u[b, 0:_L]
            u1 = rows_u[b, _L:2 * _L]
            v0 = rows_i[b, 0:_L]
            v1 = rows_i[b, _L:2 * _L]
            s = jnp.sum(u0 * v0 + u1 * v1)
            acc = jnp.where(lane == k, s, acc)
        out_v[pl.ds(b0, _L)] = acc
        return carry

    lax.fori_loop(0, _BPW // _L, group_body, 0)

    pltpu.sync_copy(out_v, out_hbm.at[pl.ds(wid * _BPW, _BPW)])


@functools.partial(jax.jit, static_argnums=())
def _pmf(u3, i3, user_emb, item_emb):
    mesh = plsc.VectorSubcoreMesh(core_axis_name="c", subcore_axis_name="s")
    return pl.kernel(
        _pmf_body,
        out_type=jax.ShapeDtypeStruct((_B,), jnp.float32),
        mesh=mesh,
        compiler_params=pltpu.CompilerParams(
            needs_layout_passes=False, use_tc_tiling_on_sc=False),
        scratch_types=[
            pltpu.VMEM((_NCHUNK, _CH), jnp.int32),
            pltpu.VMEM((_NCHUNK, _CH), jnp.int32),
            pltpu.VMEM((_BPW, _D), jnp.float32),
            pltpu.VMEM((_BPW, _D), jnp.float32),
            pltpu.VMEM((_BPW,), jnp.float32),
            pltpu.SemaphoreType.DMA,
            pltpu.SemaphoreType.DMA,
        ],
    )(u3, i3, user_emb, item_emb)


def kernel(u, i, user_emb, item_emb):
    u3 = u.astype(jnp.int32).reshape(_NW, _NCHUNK, _CH)
    i3 = i.astype(jnp.int32).reshape(_NW, _NCHUNK, _CH)
    return _pmf(u3, i3, user_emb, item_emb)
